# Initial kernel scaffold; baseline (speedup 1.0000x reference)
import jax
import jax.numpy as jnp
from jax.experimental import pallas as pl


def kernel(x, adj, W0, W1, Wss):
    raise NotImplementedError("write your pallas kernel here")


# speedup vs baseline: 1.0264x; 1.0264x over previous
import functools

import jax
import jax.numpy as jnp
from jax.experimental import pallas as pl
from jax.experimental.pallas import tpu as pltpu

_BM = 400


def _contract_t(t, w):
    return jax.lax.dot_general(
        t, w, (((1,), (1,)), ((), ())), preferred_element_type=jnp.float32
    )


def _layer1_kernel(adj_ref, x_ref, w0_ref, h_ref):
    t = jnp.dot(adj_ref[...], x_ref[...], preferred_element_type=jnp.float32)
    h_ref[...] = jnp.maximum(_contract_t(t, w0_ref[...]), 0.0)


def _layer2_kernel(adj_ref, h_ref, w1_ref, wss_ref, out_ref, xs_ref):
    t = jnp.dot(adj_ref[...], h_ref[...], preferred_element_type=jnp.float32)
    out_ref[...] = _contract_t(t, w1_ref[...])
    xs_ref[...] = _contract_t(t, wss_ref[...])


@jax.jit
def kernel(x, adj, W0, W1, Wss):
    n, d = x.shape
    bm = _BM
    grid = (n // bm,)
    adj_spec = pl.BlockSpec((bm, n), lambda i: (i, 0))
    full_spec = pl.BlockSpec((n, d), lambda i: (0, 0))
    w_spec = pl.BlockSpec(W0.shape, lambda i: (0, 0))
    row_spec = pl.BlockSpec((bm, d), lambda i: (i, 0))
    params = pltpu.CompilerParams(dimension_semantics=("parallel",))

    h = pl.pallas_call(
        _layer1_kernel,
        grid=grid,
        in_specs=[adj_spec, full_spec, w_spec],
        out_specs=row_spec,
        out_shape=jax.ShapeDtypeStruct((n, d), jnp.float32),
        compiler_params=params,
    )(adj, x, W0)

    out, xs = pl.pallas_call(
        _layer2_kernel,
        grid=grid,
        in_specs=[adj_spec, full_spec, w_spec, pl.BlockSpec(Wss.shape, lambda i: (0, 0))],
        out_specs=[row_spec, pl.BlockSpec((bm, Wss.shape[0]), lambda i: (i, 0))],
        out_shape=[
            jax.ShapeDtypeStruct((n, d), jnp.float32),
            jax.ShapeDtypeStruct((n, Wss.shape[0]), jnp.float32),
        ],
        compiler_params=params,
    )(adj, h, W1, Wss)

    return (out, xs)


# speedup vs baseline: 1.0621x; 1.0347x over previous
import jax
import jax.numpy as jnp
from jax.experimental import pallas as pl
from jax.experimental.pallas import tpu as pltpu

_BM = 400


def _contract_t(t, w):
    return jax.lax.dot_general(
        t, w, (((1,), (1,)), ((), ())), preferred_element_type=jnp.float32
    )


def _fused_kernel(adj_ref, x_ref, w0_ref, w1_ref, wss_ref, out_ref, xs_ref, h_ref):
    p = pl.program_id(0)
    i = pl.program_id(1)

    @pl.when(p == 0)
    def _phase0():
        t = jnp.dot(adj_ref[...], x_ref[...], preferred_element_type=jnp.float32)
        h_ref[pl.ds(i * _BM, _BM), :] = jnp.maximum(_contract_t(t, w0_ref[...]), 0.0)

    @pl.when(p == 1)
    def _phase1():
        t2 = jnp.dot(adj_ref[...], h_ref[...], preferred_element_type=jnp.float32)
        out_ref[...] = _contract_t(t2, w1_ref[...])
        xs_ref[...] = _contract_t(t2, wss_ref[...])


@jax.jit
def kernel(x, adj, W0, W1, Wss):
    n, d = x.shape
    bm = _BM
    ss = Wss.shape[0]
    grid = (2, n // bm)
    adj_spec = pl.BlockSpec((bm, n), lambda p, i: (i, 0))
    full_spec = pl.BlockSpec((n, d), lambda p, i: (0, 0))
    w_spec = pl.BlockSpec((d, d), lambda p, i: (0, 0))
    out_spec = pl.BlockSpec((bm, d), lambda p, i: (p * i, 0))
    xs_spec = pl.BlockSpec((bm, ss), lambda p, i: (p * i, 0))

    out, xs = pl.pallas_call(
        _fused_kernel,
        grid=grid,
        in_specs=[adj_spec, full_spec, w_spec, w_spec,
                  pl.BlockSpec((ss, d), lambda p, i: (0, 0))],
        out_specs=[out_spec, xs_spec],
        out_shape=[
            jax.ShapeDtypeStruct((n, d), jnp.float32),
            jax.ShapeDtypeStruct((n, ss), jnp.float32),
        ],
        scratch_shapes=[pltpu.VMEM((n, d), jnp.float32)],
        compiler_params=pltpu.CompilerParams(
            dimension_semantics=("arbitrary", "arbitrary")
        ),
    )(adj, x, W0, W1, Wss)

    return (out, xs)
